# baseline (speedup 1.0000x reference)
import functools

import jax
import jax.numpy as jnp
from jax import lax
from jax.experimental import pallas as pl
from jax.experimental.pallas import tpu as pltpu
from jax.experimental.pallas import tpu_sc as plsc

VOCAB = 1000000
EMBED_DIM = 32
BATCH = 4096
SEQ = 200

_INFO = plsc.get_sparse_core_info()
_NC, _NS = _INFO.num_cores, _INFO.num_subcores
_NW = _NC * _NS

_TBLK = 2048
_NTB = (VOCAB + _TBLK - 1) // _TBLK
_QROWS4 = _NTB * (_TBLK // 4)
_QROWS = _QROWS4 * 4

_CBB = 512
_N = BATCH * SEQ
_TOK_PER_W = _N // _NW
_CHUNKS_PER_W = _TOK_PER_W // _CBB


def _make_tpose_tc():
    @functools.partial(
        pl.pallas_call,
        grid=(_NTB,),
        in_specs=[pl.BlockSpec((EMBED_DIM, _TBLK), lambda i: (0, i))],
        out_specs=pl.BlockSpec((_TBLK // 4, 128), lambda i: (i, 0)),
        out_shape=jax.ShapeDtypeStruct((_QROWS4, 128), jnp.float32),
    )
    def tpose_tc(in_ref, out_ref):
        x = in_ref[...]
        for a in range(4):
            out_ref[:, 32 * a:32 * (a + 1)] = x[:, 512 * a:512 * (a + 1)].T

    return tpose_tc


def _make_retile_tc():
    @functools.partial(
        pl.pallas_call,
        grid=(SEQ,),
        in_specs=[pl.BlockSpec((1, 1024, 128), lambda s: (s, 0, 0))],
        out_specs=pl.BlockSpec((1, EMBED_DIM, BATCH), lambda s: (s, 0, 0)),
        out_shape=jax.ShapeDtypeStruct((SEQ, EMBED_DIM, BATCH), jnp.float32),
    )
    def retile_tc(in_ref, out_ref):
        out_ref[0] = in_ref[0].reshape(EMBED_DIM, BATCH)

    return retile_tc


def _make_gather():
    mesh = plsc.VectorSubcoreMesh(core_axis_name="c", subcore_axis_name="s")

    @functools.partial(
        pl.kernel,
        out_type=jax.ShapeDtypeStruct((SEQ, EMBED_DIM, BATCH), jnp.float32),
        mesh=mesh,
        scratch_types=[
            pltpu.VMEM((_CBB,), jnp.int32),
            pltpu.VMEM((_CBB,), jnp.int32),
            pltpu.VMEM((_CBB, EMBED_DIM), jnp.float32),
            pltpu.VMEM((EMBED_DIM, _CBB), jnp.float32),
            pltpu.SemaphoreType.DMA,
        ],
        compiler_params=pltpu.CompilerParams(
            use_tc_tiling_on_sc=False, needs_layout_passes=False),
    )
    def gath(idx_hbm, tq_hbm, out, idx_v, r_v, g_v, out_v, sem):
        wid = lax.axis_index("s") * _NC + lax.axis_index("c")
        base = wid * _TOK_PER_W
        i16 = lax.iota(jnp.int32, 16)

        def body(u, carry):
            t0 = pl.multiple_of(base + u * _CBB, _CBB)
            s = lax.shift_right_logical(t0, 12)
            b0 = pl.multiple_of(t0 & (BATCH - 1), _CBB)
            pltpu.sync_copy(idx_hbm.at[pl.ds(t0, _CBB)], idx_v)

            def pre(g, c2):
                v = idx_v[pl.ds(16 * g, 16)]
                hi = lax.shift_left(lax.shift_right_logical(v, 11), 11)
                r_v[pl.ds(16 * g, 16)] = (
                    hi + lax.shift_left(v & 511, 2)
                    + (lax.shift_right_logical(v, 9) & 3))
                return c2

            lax.fori_loop(0, _CBB // 16, pre, 0)
            pltpu.async_copy(tq_hbm.at[r_v], g_v, sem).wait()

            def ext(tg, c2):
                rows = 16 * tg + i16
                for d in range(EMBED_DIM):
                    out_v[d, pl.ds(16 * tg, 16)] = plsc.load_gather(
                        g_v, [rows, jnp.full((16,), d, jnp.int32)])
                return c2

            lax.fori_loop(0, _CBB // 16, ext, 0)
            pltpu.sync_copy(out_v, out.at[s, :, pl.ds(b0, _CBB)])
            return carry

        lax.fori_loop(0, _CHUNKS_PER_W, body, 0)

    return gath


_TPOSE_TC = _make_tpose_tc()
_RETILE_TC = _make_retile_tc()
_GATH = _make_gather()


def kernel(indices, table):
    tq = _TPOSE_TC(table.T)
    tqv = tq.reshape(-1).reshape(_QROWS, EMBED_DIM)
    idx_flat = indices.T.reshape(-1).astype(jnp.int32)
    out_lin = _GATH(idx_flat, tqv)
    out_phys = _RETILE_TC(out_lin.reshape(SEQ, 1024, 128))
    return out_phys.transpose(2, 0, 1)
